# baseline (speedup 1.0000x reference)
import jax
import jax.numpy as jnp
from jax import lax
from jax.experimental import pallas as pl
from jax.experimental.pallas import tpu as pltpu
from jax.experimental.pallas import tpu_sc as plsc

_T = 26
_V = 100000
_D = 32
_B = 81920
_NC = 2
_NS = 16
_NW = _NC * _NS
_CHUNK = 128
_ROWS = _B // _NW // _CHUNK
_VFULL = _V // _CHUNK
_VTAIL = _V - _VFULL * _CHUNK


def _detile_body(tt_hbm, tails_hbm, s_hbm, inb, outb, tailb, sem, osem):
    wid = lax.axis_index("s") * _NC + lax.axis_index("c")
    iota = lax.iota(jnp.int32, 16)

    qruns = [(vl0 * 16 + iota) >> 2 for vl0 in range(8)]
    k32runs = [((vl0 * 16 + iota) & 3) * 32 for vl0 in range(8)]

    nk = jnp.where(wid < _VFULL - 24 * _NW, 25, 24)
    nslots = _T * nk

    def fire_in(t, c, b):
        pltpu.async_copy(
            tt_hbm.at[t, :, pl.ds(c * _CHUNK, _CHUNK)],
            inb.at[b, :, pl.ds(0, _CHUNK)],
            sem,
        )

    def wait_in():
        pltpu.make_async_copy(
            tt_hbm.at[0, :, pl.ds(0, _CHUNK)],
            inb.at[0, :, pl.ds(0, _CHUNK)],
            sem,
        ).wait()

    def wait_out():
        pltpu.make_async_copy(
            outb.at[0, :, pl.ds(0, _CHUNK)], s_hbm.at[0, pl.ds(0, 32)], osem
        ).wait()

    def advance(t, k):
        wrap = k + 1 >= nk
        return jnp.where(wrap, t + 1, t), jnp.where(wrap, 0, k + 1)

    def compute(t, c, bin_, bout):
        def one_d(d2, carry):
            for dd in range(2):
                d = d2 * 2 + dd
                dv = jnp.full((16,), d, jnp.int32)
                for vl0 in range(8):
                    x = inb[bin_, d, pl.ds(vl0 * 16, 16)]
                    plsc.store_scatter(
                        outb.at[bout], [qruns[vl0], k32runs[vl0] + dv], x
                    )
            return carry

        lax.fori_loop(0, 16, one_d, 0)
        pltpu.async_copy(
            outb.at[bout, :, pl.ds(0, _CHUNK)],
            s_hbm.at[t, pl.ds(c * 32, 32)],
            osem,
        )

    def slot_step(i, carry):
        tc, kc, tp, kp = carry

        @pl.when(i + 2 < nslots)
        def _():
            fire_in(tp, wid + kp * _NW, (i + 2) % 3)

        wait_in()

        @pl.when(i >= 2)
        def _():
            wait_out()

        compute(tc, wid + kc * _NW, i % 3, i % 2)
        tc2, kc2 = advance(tc, kc)
        tp2, kp2 = advance(tp, kp)
        return (tc2, kc2, tp2, kp2)

    fire_in(0, wid, 0)

    @pl.when(nslots > 1)
    def _():
        t1, k1_ = advance(jnp.int32(0), jnp.int32(0))
        fire_in(t1, wid + k1_ * _NW, 1)

    t2, k2_ = advance(jnp.int32(0), jnp.int32(0))
    t2, k2_ = advance(t2, k2_)
    lax.fori_loop(
        0,
        nslots,
        slot_step,
        (jnp.int32(0), jnp.int32(0), t2, k2_),
    )
    wait_out()
    wait_out()

    @pl.when(wid < _T)
    def _():
        t = wid
        pltpu.sync_copy(tails_hbm.at[t], tailb)
        pltpu.sync_copy(tailb, s_hbm.at[t, pl.ds(_VFULL * 32, 8)])


def _gather_body(values_hbm, s_hbm, o_hbm, idx_v, rows_v, outb, gsem, osem):
    wid = lax.axis_index("s") * _NC + lax.axis_index("c")
    iota = lax.iota(jnp.int32, 16)
    gruns = [(k * 16 + iota) >> 3 for k in range(2)]
    rruns = [(k * 16 + iota) & 7 for k in range(2)]

    def table_step(t, carry):
        pltpu.sync_copy(values_hbm.at[t, wid], idx_v)
        for j in range(_ROWS):
            pltpu.async_copy(s_hbm.at[t].at[idx_v.at[j]], rows_v.at[j], gsem)

        def wait_out():
            pltpu.make_async_copy(
                outb.at[0, :, :, pl.ds(0, _CHUNK)], o_hbm.at[0, :, 0], osem
            ).wait()

        def chunk_step(j, c2):
            pltpu.make_async_copy(
                s_hbm.at[t, pl.ds(0, _CHUNK)], rows_v.at[0], gsem
            ).wait()

            @pl.when(j >= 2)
            def _():
                wait_out()

            def one_m(m, c3):
                mv = jnp.full((16,), m, jnp.int32)
                for k in range(2):
                    x = rows_v[j, m, pl.ds(k * 16, 16)]
                    plsc.store_scatter(outb.at[j % 2], [gruns[k], rruns[k], mv], x)
                return c3

            lax.fori_loop(0, _CHUNK, one_m, 0)
            pltpu.async_copy(
                outb.at[j % 2, :, :, pl.ds(0, _CHUNK)],
                o_hbm.at[t, :, wid * _ROWS + j],
                osem,
            )
            return c2

        lax.fori_loop(0, _ROWS, chunk_step, 0)
        wait_out()
        wait_out()
        return carry

    lax.fori_loop(0, _T, table_step, 0)


def kernel(values, tables):
    mesh = plsc.VectorSubcoreMesh(core_axis_name="c", subcore_axis_name="s")
    tt = tables.transpose(0, 2, 1)
    tails5 = tables[:, _VFULL * _CHUNK :, :].reshape(_T, 8, _CHUNK)
    s = pl.kernel(
        _detile_body,
        out_type=jax.ShapeDtypeStruct((_T, _V // 4, _CHUNK), jnp.float32),
        mesh=mesh,
        scratch_types=[
            pltpu.VMEM((3, _D, _CHUNK + 1), jnp.float32),
            pltpu.VMEM((2, 32, _CHUNK + 4), jnp.float32),
            pltpu.VMEM((8, _CHUNK), jnp.float32),
            pltpu.SemaphoreType.DMA,
            pltpu.SemaphoreType.DMA,
        ],
        compiler_params=pltpu.CompilerParams(
            use_tc_tiling_on_sc=True, needs_layout_passes=False
        ),
    )(tt, tails5)

    values_r = values.reshape(_T, _NW, _ROWS, _CHUNK)
    o5 = pl.kernel(
        _gather_body,
        out_type=jax.ShapeDtypeStruct((_T, 4, _NW * _ROWS, 8, _CHUNK), jnp.float32),
        mesh=mesh,
        scratch_types=[
            pltpu.VMEM((_ROWS, _CHUNK), jnp.int32),
            pltpu.VMEM((_ROWS, _CHUNK, _D), jnp.float32),
            pltpu.VMEM((2, 4, 8, _CHUNK + 1), jnp.float32),
            pltpu.SemaphoreType.DMA,
            pltpu.SemaphoreType.DMA,
        ],
        compiler_params=pltpu.CompilerParams(
            use_tc_tiling_on_sc=False, needs_layout_passes=False
        ),
    )(values_r, s.reshape(_T, _V, _D))
    return o5.transpose(0, 2, 4, 1, 3).reshape(_T, _B, _D)


# speedup vs baseline: 1.0165x; 1.0165x over previous
import jax
import jax.numpy as jnp
from jax import lax
from jax.experimental import pallas as pl
from jax.experimental.pallas import tpu as pltpu
from jax.experimental.pallas import tpu_sc as plsc

_T = 26
_V = 100000
_D = 32
_B = 81920
_NC = 2
_NS = 16
_NW = _NC * _NS
_CHUNK = 128
_ROWS = _B // _NW // _CHUNK
_VFULL = _V // _CHUNK
_VTAIL = _V - _VFULL * _CHUNK


def _detile_body(tt_hbm, tails_hbm, s_hbm, inb, outb, tailb, sem, osem):
    wid = lax.axis_index("s") * _NC + lax.axis_index("c")
    iota = lax.iota(jnp.int32, 16)

    qruns = [(vl0 * 16 + iota) >> 2 for vl0 in range(8)]
    k32runs = [((vl0 * 16 + iota) & 3) * 32 for vl0 in range(8)]

    nk = jnp.where(wid < _VFULL - 24 * _NW, 25, 24)
    nslots = _T * nk

    def fire_in(t, c, b):
        pltpu.async_copy(
            tt_hbm.at[t, :, pl.ds(c * _CHUNK, _CHUNK)],
            inb.at[b, :, pl.ds(0, _CHUNK)],
            sem,
        )

    def wait_in():
        pltpu.make_async_copy(
            tt_hbm.at[0, :, pl.ds(0, _CHUNK)],
            inb.at[0, :, pl.ds(0, _CHUNK)],
            sem,
        ).wait()

    def wait_out():
        pltpu.make_async_copy(
            outb.at[0, :, pl.ds(0, _CHUNK)], s_hbm.at[0, pl.ds(0, 32)], osem
        ).wait()

    def advance(t, k):
        wrap = k + 1 >= nk
        return jnp.where(wrap, t + 1, t), jnp.where(wrap, 0, k + 1)

    def compute(t, c, bin_, bout):
        def one_d(d2, carry):
            for dd in range(2):
                d = d2 * 2 + dd
                dv = jnp.full((16,), d, jnp.int32)
                for vl0 in range(8):
                    x = inb[bin_, d, pl.ds(vl0 * 16, 16)]
                    plsc.store_scatter(
                        outb.at[bout], [qruns[vl0], k32runs[vl0] + dv], x
                    )
            return carry

        lax.fori_loop(0, 16, one_d, 0)
        pltpu.async_copy(
            outb.at[bout, :, pl.ds(0, _CHUNK)],
            s_hbm.at[t, pl.ds(c * 32, 32)],
            osem,
        )

    def slot_step(i, carry):
        tc, kc, tp, kp = carry

        @pl.when(i + 2 < nslots)
        def _():
            fire_in(tp, wid + kp * _NW, (i + 2) % 3)

        wait_in()

        @pl.when(i >= 2)
        def _():
            wait_out()

        compute(tc, wid + kc * _NW, i % 3, i % 2)
        tc2, kc2 = advance(tc, kc)
        tp2, kp2 = advance(tp, kp)
        return (tc2, kc2, tp2, kp2)

    fire_in(0, wid, 0)

    @pl.when(nslots > 1)
    def _():
        t1, k1_ = advance(jnp.int32(0), jnp.int32(0))
        fire_in(t1, wid + k1_ * _NW, 1)

    t2, k2_ = advance(jnp.int32(0), jnp.int32(0))
    t2, k2_ = advance(t2, k2_)
    lax.fori_loop(
        0,
        nslots,
        slot_step,
        (jnp.int32(0), jnp.int32(0), t2, k2_),
    )
    wait_out()
    wait_out()

    @pl.when(wid < _T)
    def _():
        t = wid
        pltpu.sync_copy(tails_hbm.at[t], tailb)
        pltpu.sync_copy(tailb, s_hbm.at[t, pl.ds(_VFULL * 32, 8)])


def _gather_body(values_hbm, s_hbm, o_hbm, idx_v, rows_v, outb, gsem, osem):
    wid = lax.axis_index("s") * _NC + lax.axis_index("c")
    iota = lax.iota(jnp.int32, 16)
    gruns = [(k * 16 + iota) >> 3 for k in range(2)]
    rruns = [(k * 16 + iota) & 7 for k in range(2)]

    def table_step(t, carry):
        pltpu.sync_copy(values_hbm.at[t, wid], idx_v)
        for j in range(_ROWS):
            pltpu.async_copy(s_hbm.at[t].at[idx_v.at[j]], rows_v.at[j], gsem)

        def wait_out():
            pltpu.make_async_copy(
                outb.at[0, :, :, pl.ds(0, _CHUNK)], o_hbm.at[0, :, 0], osem
            ).wait()

        def chunk_step(j, c2):
            pltpu.make_async_copy(
                s_hbm.at[t, pl.ds(0, _CHUNK)], rows_v.at[0], gsem
            ).wait()

            @pl.when(j >= 2)
            def _():
                wait_out()

            def one_m(m2, c3):
                for dm in range(2):
                    m = m2 * 2 + dm
                    mv = jnp.full((16,), m, jnp.int32)
                    for k in range(2):
                        x = rows_v[j, m, pl.ds(k * 16, 16)]
                        plsc.store_scatter(
                            outb.at[j % 2], [gruns[k], rruns[k], mv], x
                        )
                return c3

            lax.fori_loop(0, _CHUNK // 2, one_m, 0)
            pltpu.async_copy(
                outb.at[j % 2, :, :, pl.ds(0, _CHUNK)],
                o_hbm.at[t, :, wid * _ROWS + j],
                osem,
            )
            return c2

        lax.fori_loop(0, _ROWS, chunk_step, 0)
        wait_out()
        wait_out()
        return carry

    lax.fori_loop(0, _T, table_step, 0)


def kernel(values, tables):
    mesh = plsc.VectorSubcoreMesh(core_axis_name="c", subcore_axis_name="s")
    tt = tables.transpose(0, 2, 1)
    tails5 = tables[:, _VFULL * _CHUNK :, :].reshape(_T, 8, _CHUNK)
    s = pl.kernel(
        _detile_body,
        out_type=jax.ShapeDtypeStruct((_T, _V // 4, _CHUNK), jnp.float32),
        mesh=mesh,
        scratch_types=[
            pltpu.VMEM((3, _D, _CHUNK + 1), jnp.float32),
            pltpu.VMEM((2, 32, _CHUNK + 4), jnp.float32),
            pltpu.VMEM((8, _CHUNK), jnp.float32),
            pltpu.SemaphoreType.DMA,
            pltpu.SemaphoreType.DMA,
        ],
        compiler_params=pltpu.CompilerParams(
            use_tc_tiling_on_sc=True, needs_layout_passes=False
        ),
    )(tt, tails5)

    values_r = values.reshape(_T, _NW, _ROWS, _CHUNK)
    o5 = pl.kernel(
        _gather_body,
        out_type=jax.ShapeDtypeStruct((_T, 4, _NW * _ROWS, 8, _CHUNK), jnp.float32),
        mesh=mesh,
        scratch_types=[
            pltpu.VMEM((_ROWS, _CHUNK), jnp.int32),
            pltpu.VMEM((_ROWS, _CHUNK, _D), jnp.float32),
            pltpu.VMEM((2, 4, 8, _CHUNK + 1), jnp.float32),
            pltpu.SemaphoreType.DMA,
            pltpu.SemaphoreType.DMA,
        ],
        compiler_params=pltpu.CompilerParams(
            use_tc_tiling_on_sc=False, needs_layout_passes=False
        ),
    )(values_r, s.reshape(_T, _V, _D))
    return o5.transpose(0, 2, 4, 1, 3).reshape(_T, _B, _D)
